# baseline (speedup 1.0000x reference)
import functools

import jax
import jax.numpy as jnp
from jax import lax
from jax.experimental import pallas as pl
from jax.experimental.pallas import tpu as pltpu
from jax.experimental.pallas import tpu_sc as plsc

_EPS = 1e-12
_NCHUNK = 8
_T = 128


def _sc_gather(table, idx):
    n = idx.shape[0]
    d = table.shape[1]
    n_cores, n_subcores = 2, 16
    n_workers = n_cores * n_subcores
    per_w = n // n_workers
    ch = min(64, per_w)
    n_ch = per_w // ch
    mesh = plsc.VectorSubcoreMesh(core_axis_name="c", subcore_axis_name="s")

    @functools.partial(
        pl.kernel,
        mesh=mesh,
        out_type=jax.ShapeDtypeStruct((n, d), jnp.float32),
        scratch_types=[
            pltpu.VMEM((per_w,), jnp.int32),
            pltpu.VMEM((ch, d), jnp.float32),
            pltpu.VMEM((ch, d), jnp.float32),
            pltpu.SemaphoreType.DMA,
            pltpu.SemaphoreType.DMA,
        ],
    )
    def gather_kernel(table_hbm, idx_hbm, out_hbm, idx_v, buf0, buf1, sem0, sem1):
        wid = lax.axis_index("s") * n_cores + lax.axis_index("c")
        base = wid * per_w
        pltpu.sync_copy(idx_hbm.at[pl.ds(base, per_w)], idx_v)
        bufs = (buf0, buf1)
        sems = (sem0, sem1)
        cur = pltpu.async_copy(table_hbm.at[idx_v.at[pl.ds(0, ch)]], buf0, sem0)
        for c in range(n_ch):
            nxt = None
            if c + 1 < n_ch:
                nxt = pltpu.async_copy(
                    table_hbm.at[idx_v.at[pl.ds((c + 1) * ch, ch)]],
                    bufs[(c + 1) % 2],
                    sems[(c + 1) % 2],
                )
            cur.wait()
            pltpu.sync_copy(bufs[c % 2], out_hbm.at[pl.ds(base + c * ch, ch)])
            cur = nxt

    return gather_kernel(table, idx)


def _tc_body(tt_ref, at_ref, g_ref, pos_ref, te_ref, ae_ref, ga_ref, be_ref,
             o_ref):
    x = g_ref[...] + pos_ref[...][None]
    tt = tt_ref[:, 0, :][..., None]
    at = at_ref[:, 0, :][..., None]
    x = x + jnp.where(tt == 1, te_ref[1], te_ref[0])
    acc = jnp.zeros_like(x)
    for k in range(5):
        acc = acc + jnp.where(at == k, ae_ref[k], 0.0)
    x = x + acc
    mu = jnp.mean(x, axis=-1, keepdims=True)
    xc = x - mu
    var = jnp.mean(xc * xc, axis=-1, keepdims=True)
    o_ref[...] = xc * lax.rsqrt(var + _EPS) * ga_ref[0] + be_ref[0]


def _tc_body_aliased(prev_ref, *refs):
    del prev_ref
    _tc_body(*refs)


def _tc_finish_chunk(c, prev_out, gathered_c, pos_emb, tt_c, at_c, type_emb,
                     ans_emb, gamma2, beta2, b, s, d, sc):
    nsteps = sc // _T
    base = c * nsteps
    in_specs = [
        pl.BlockSpec((b, 1, _T), lambda j: (0, 0, j)),
        pl.BlockSpec((b, 1, _T), lambda j: (0, 0, j)),
        pl.BlockSpec((b, _T, d), lambda j: (0, j, 0)),
        pl.BlockSpec((_T, d), lambda j, _b=base: (_b + j, 0)),
        pl.BlockSpec(type_emb.shape, lambda j: (0, 0)),
        pl.BlockSpec(ans_emb.shape, lambda j: (0, 0)),
        pl.BlockSpec((1, d), lambda j: (0, 0)),
        pl.BlockSpec((1, d), lambda j: (0, 0)),
    ]
    out_spec = pl.BlockSpec((b, _T, d), lambda j, _b=base: (0, _b + j, 0))
    args = (tt_c, at_c, gathered_c.reshape(b, sc, d), pos_emb, type_emb,
            ans_emb, gamma2, beta2)
    if prev_out is None:
        return pl.pallas_call(
            _tc_body,
            grid=(nsteps,),
            in_specs=in_specs,
            out_specs=out_spec,
            out_shape=jax.ShapeDtypeStruct((b, s, d), jnp.float32),
        )(*args)
    return pl.pallas_call(
        _tc_body_aliased,
        grid=(nsteps,),
        in_specs=[pl.BlockSpec(memory_space=pl.ANY)] + in_specs,
        out_specs=out_spec,
        out_shape=jax.ShapeDtypeStruct((b, s, d), jnp.float32),
        input_output_aliases={0: 0},
    )(prev_out, *args)


def kernel(input_ids, token_type_ids, ans_type_ids, word_emb, pos_emb, type_emb,
           ans_emb, gamma, beta):
    b, s = input_ids.shape
    d = word_emb.shape[1]
    sc = s // _NCHUNK
    ids_c = input_ids.reshape(b, _NCHUNK, sc).transpose(1, 0, 2)
    tt_c = token_type_ids.reshape(b, _NCHUNK, 1, sc).transpose(1, 0, 2, 3)
    at_c = ans_type_ids.reshape(b, _NCHUNK, 1, sc).transpose(1, 0, 2, 3)
    gamma2 = gamma.reshape(1, d)
    beta2 = beta.reshape(1, d)
    gathered = [
        _sc_gather(word_emb, ids_c[c].reshape(b * sc).astype(jnp.int32))
        for c in range(_NCHUNK)
    ]
    out = None
    for c in range(_NCHUNK):
        out = _tc_finish_chunk(c, out, gathered[c], pos_emb,
                               tt_c[c].astype(jnp.int32),
                               at_c[c].astype(jnp.int32),
                               type_emb, ans_emb, gamma2, beta2, b, s, d, sc)
    return out


# speedup vs baseline: 1.2652x; 1.2652x over previous
import functools

import jax
import jax.numpy as jnp
from jax import lax
from jax.experimental import pallas as pl
from jax.experimental.pallas import tpu as pltpu
from jax.experimental.pallas import tpu_sc as plsc

_EPS = 1e-12
_NCHUNK = 2
_T = 128


def _sc_gather(table, idx):
    n = idx.shape[0]
    d = table.shape[1]
    n_cores, n_subcores = 2, 16
    n_workers = n_cores * n_subcores
    per_w = n // n_workers
    ch = min(64, per_w)
    n_ch = per_w // ch
    mesh = plsc.VectorSubcoreMesh(core_axis_name="c", subcore_axis_name="s")

    @functools.partial(
        pl.kernel,
        mesh=mesh,
        out_type=jax.ShapeDtypeStruct((n, d), jnp.float32),
        scratch_types=[
            pltpu.VMEM((per_w,), jnp.int32),
            pltpu.VMEM((ch, d), jnp.float32),
            pltpu.VMEM((ch, d), jnp.float32),
            pltpu.SemaphoreType.DMA,
            pltpu.SemaphoreType.DMA,
        ],
    )
    def gather_kernel(table_hbm, idx_hbm, out_hbm, idx_v, buf0, buf1, sem0, sem1):
        wid = lax.axis_index("s") * n_cores + lax.axis_index("c")
        base = wid * per_w
        pltpu.sync_copy(idx_hbm.at[pl.ds(base, per_w)], idx_v)
        bufs = (buf0, buf1)
        sems = (sem0, sem1)
        cur = pltpu.async_copy(table_hbm.at[idx_v.at[pl.ds(0, ch)]], buf0, sem0)
        for c in range(n_ch):
            nxt = None
            if c + 1 < n_ch:
                nxt = pltpu.async_copy(
                    table_hbm.at[idx_v.at[pl.ds((c + 1) * ch, ch)]],
                    bufs[(c + 1) % 2],
                    sems[(c + 1) % 2],
                )
            cur.wait()
            pltpu.sync_copy(bufs[c % 2], out_hbm.at[pl.ds(base + c * ch, ch)])
            cur = nxt

    return gather_kernel(table, idx)


def _tc_body(tt_ref, at_ref, g_ref, pos_ref, te_ref, ae_ref, ga_ref, be_ref,
             o_ref):
    x = g_ref[...] + pos_ref[...][None]
    tt = tt_ref[:, 0, :][..., None]
    at = at_ref[:, 0, :][..., None]
    x = x + jnp.where(tt == 1, te_ref[1], te_ref[0])
    acc = jnp.zeros_like(x)
    for k in range(5):
        acc = acc + jnp.where(at == k, ae_ref[k], 0.0)
    x = x + acc
    mu = jnp.mean(x, axis=-1, keepdims=True)
    xc = x - mu
    var = jnp.mean(xc * xc, axis=-1, keepdims=True)
    o_ref[...] = xc * lax.rsqrt(var + _EPS) * ga_ref[0] + be_ref[0]


def _tc_body_aliased(prev_ref, *refs):
    del prev_ref
    _tc_body(*refs)


def _tc_finish_chunk(c, prev_out, gathered_c, pos_emb, tt_c, at_c, type_emb,
                     ans_emb, gamma2, beta2, b, s, d, sc):
    nsteps = sc // _T
    base = c * nsteps
    in_specs = [
        pl.BlockSpec((b, 1, _T), lambda j: (0, 0, j)),
        pl.BlockSpec((b, 1, _T), lambda j: (0, 0, j)),
        pl.BlockSpec((b, _T, d), lambda j: (0, j, 0)),
        pl.BlockSpec((_T, d), lambda j, _b=base: (_b + j, 0)),
        pl.BlockSpec(type_emb.shape, lambda j: (0, 0)),
        pl.BlockSpec(ans_emb.shape, lambda j: (0, 0)),
        pl.BlockSpec((1, d), lambda j: (0, 0)),
        pl.BlockSpec((1, d), lambda j: (0, 0)),
    ]
    out_spec = pl.BlockSpec((b, _T, d), lambda j, _b=base: (0, _b + j, 0))
    args = (tt_c, at_c, gathered_c.reshape(b, sc, d), pos_emb, type_emb,
            ans_emb, gamma2, beta2)
    if prev_out is None:
        return pl.pallas_call(
            _tc_body,
            grid=(nsteps,),
            in_specs=in_specs,
            out_specs=out_spec,
            out_shape=jax.ShapeDtypeStruct((b, s, d), jnp.float32),
        )(*args)
    return pl.pallas_call(
        _tc_body_aliased,
        grid=(nsteps,),
        in_specs=[pl.BlockSpec(memory_space=pl.ANY)] + in_specs,
        out_specs=out_spec,
        out_shape=jax.ShapeDtypeStruct((b, s, d), jnp.float32),
        input_output_aliases={0: 0},
    )(prev_out, *args)


def kernel(input_ids, token_type_ids, ans_type_ids, word_emb, pos_emb, type_emb,
           ans_emb, gamma, beta):
    b, s = input_ids.shape
    d = word_emb.shape[1]
    sc = s // _NCHUNK
    ids_c = input_ids.reshape(b, _NCHUNK, sc).transpose(1, 0, 2)
    tt_c = token_type_ids.reshape(b, _NCHUNK, 1, sc).transpose(1, 0, 2, 3)
    at_c = ans_type_ids.reshape(b, _NCHUNK, 1, sc).transpose(1, 0, 2, 3)
    gamma2 = gamma.reshape(1, d)
    beta2 = beta.reshape(1, d)
    gathered = [
        _sc_gather(word_emb, ids_c[c].reshape(b * sc).astype(jnp.int32))
        for c in range(_NCHUNK)
    ]
    out = None
    for c in range(_NCHUNK):
        out = _tc_finish_chunk(c, out, gathered[c], pos_emb,
                               tt_c[c].astype(jnp.int32),
                               at_c[c].astype(jnp.int32),
                               type_emb, ans_emb, gamma2, beta2, b, s, d, sc)
    return out


# speedup vs baseline: 1.2674x; 1.0018x over previous
import functools

import jax
import jax.numpy as jnp
from jax import lax
from jax.experimental import pallas as pl
from jax.experimental.pallas import tpu as pltpu
from jax.experimental.pallas import tpu_sc as plsc

_EPS = 1e-12
_NCHUNK = 2
_T = 128


def _sc_gather(table, idx):
    n = idx.shape[0]
    d = table.shape[1]
    n_cores, n_subcores = 2, 16
    n_workers = n_cores * n_subcores
    per_w = n // n_workers
    ch = min(64, per_w)
    n_ch = per_w // ch
    mesh = plsc.VectorSubcoreMesh(core_axis_name="c", subcore_axis_name="s")

    @functools.partial(
        pl.kernel,
        mesh=mesh,
        out_type=jax.ShapeDtypeStruct((n, d), jnp.float32),
        scratch_types=[
            pltpu.VMEM((per_w,), jnp.int32),
            pltpu.VMEM((ch, d), jnp.float32),
            pltpu.VMEM((ch, d), jnp.float32),
            pltpu.SemaphoreType.DMA,
            pltpu.SemaphoreType.DMA,
            pltpu.SemaphoreType.DMA,
            pltpu.SemaphoreType.DMA,
        ],
    )
    def gather_kernel(table_hbm, idx_hbm, out_hbm, idx_v, buf0, buf1,
                      gs0, gs1, ws0, ws1):
        wid = lax.axis_index("s") * n_cores + lax.axis_index("c")
        base = wid * per_w
        pltpu.sync_copy(idx_hbm.at[pl.ds(base, per_w)], idx_v)
        bufs = (buf0, buf1)
        gsems = (gs0, gs1)
        wsems = (ws0, ws1)
        gathers = [None] * n_ch
        for c in range(min(2, n_ch)):
            gathers[c] = pltpu.async_copy(
                table_hbm.at[idx_v.at[pl.ds(c * ch, ch)]], bufs[c % 2],
                gsems[c % 2])
        writes = []
        for c in range(n_ch):
            gathers[c].wait()
            if len(writes) >= 2:
                writes[c - 2].wait()
            writes.append(pltpu.async_copy(
                bufs[c % 2], out_hbm.at[pl.ds(base + c * ch, ch)],
                wsems[c % 2]))
            if c + 2 < n_ch:
                writes[c].wait()
                gathers[c + 2] = pltpu.async_copy(
                    table_hbm.at[idx_v.at[pl.ds((c + 2) * ch, ch)]],
                    bufs[c % 2], gsems[c % 2])
        for w in writes[-2:]:
            w.wait()

    return gather_kernel(table, idx)


def _tc_body(tt_ref, at_ref, g_ref, pos_ref, te_ref, ae_ref, ga_ref, be_ref,
             o_ref):
    x = g_ref[...] + pos_ref[...][None]
    tt = tt_ref[:, 0, :][..., None]
    at = at_ref[:, 0, :][..., None]
    x = x + jnp.where(tt == 1, te_ref[1], te_ref[0])
    acc = jnp.zeros_like(x)
    for k in range(5):
        acc = acc + jnp.where(at == k, ae_ref[k], 0.0)
    x = x + acc
    mu = jnp.mean(x, axis=-1, keepdims=True)
    xc = x - mu
    var = jnp.mean(xc * xc, axis=-1, keepdims=True)
    o_ref[...] = xc * lax.rsqrt(var + _EPS) * ga_ref[0] + be_ref[0]


def _tc_body_aliased(prev_ref, *refs):
    del prev_ref
    _tc_body(*refs)


def _tc_finish_chunk(c, prev_out, gathered_c, pos_emb, tt_c, at_c, type_emb,
                     ans_emb, gamma2, beta2, b, s, d, sc):
    nsteps = sc // _T
    base = c * nsteps
    in_specs = [
        pl.BlockSpec((b, 1, _T), lambda j: (0, 0, j)),
        pl.BlockSpec((b, 1, _T), lambda j: (0, 0, j)),
        pl.BlockSpec((b, _T, d), lambda j: (0, j, 0)),
        pl.BlockSpec((_T, d), lambda j, _b=base: (_b + j, 0)),
        pl.BlockSpec(type_emb.shape, lambda j: (0, 0)),
        pl.BlockSpec(ans_emb.shape, lambda j: (0, 0)),
        pl.BlockSpec((1, d), lambda j: (0, 0)),
        pl.BlockSpec((1, d), lambda j: (0, 0)),
    ]
    out_spec = pl.BlockSpec((b, _T, d), lambda j, _b=base: (0, _b + j, 0))
    args = (tt_c, at_c, gathered_c.reshape(b, sc, d), pos_emb, type_emb,
            ans_emb, gamma2, beta2)
    if prev_out is None:
        return pl.pallas_call(
            _tc_body,
            grid=(nsteps,),
            in_specs=in_specs,
            out_specs=out_spec,
            out_shape=jax.ShapeDtypeStruct((b, s, d), jnp.float32),
        )(*args)
    return pl.pallas_call(
        _tc_body_aliased,
        grid=(nsteps,),
        in_specs=[pl.BlockSpec(memory_space=pl.ANY)] + in_specs,
        out_specs=out_spec,
        out_shape=jax.ShapeDtypeStruct((b, s, d), jnp.float32),
        input_output_aliases={0: 0},
    )(prev_out, *args)


def kernel(input_ids, token_type_ids, ans_type_ids, word_emb, pos_emb, type_emb,
           ans_emb, gamma, beta):
    b, s = input_ids.shape
    d = word_emb.shape[1]
    sc = s // _NCHUNK
    ids_c = input_ids.reshape(b, _NCHUNK, sc).transpose(1, 0, 2)
    tt_c = token_type_ids.reshape(b, _NCHUNK, 1, sc).transpose(1, 0, 2, 3)
    at_c = ans_type_ids.reshape(b, _NCHUNK, 1, sc).transpose(1, 0, 2, 3)
    gamma2 = gamma.reshape(1, d)
    beta2 = beta.reshape(1, d)
    gathered = [
        _sc_gather(word_emb, ids_c[c].reshape(b * sc).astype(jnp.int32))
        for c in range(_NCHUNK)
    ]
    out = None
    for c in range(_NCHUNK):
        out = _tc_finish_chunk(c, out, gathered[c], pos_emb,
                               tt_c[c].astype(jnp.int32),
                               at_c[c].astype(jnp.int32),
                               type_emb, ans_emb, gamma2, beta2, b, s, d, sc)
    return out


# speedup vs baseline: 1.2886x; 1.0167x over previous
import functools

import jax
import jax.numpy as jnp
from jax import lax
from jax.experimental import pallas as pl
from jax.experimental.pallas import tpu as pltpu
from jax.experimental.pallas import tpu_sc as plsc

_EPS = 1e-12
_NCHUNK = 2
_T = 128


def _sc_gather(table, idx):
    n = idx.shape[0]
    d = table.shape[1]
    n_cores, n_subcores = 2, 16
    n_workers = n_cores * n_subcores
    per_w = n // n_workers
    ch = min(128, per_w)
    n_ch = per_w // ch
    mesh = plsc.VectorSubcoreMesh(core_axis_name="c", subcore_axis_name="s")

    @functools.partial(
        pl.kernel,
        mesh=mesh,
        out_type=jax.ShapeDtypeStruct((n, d), jnp.float32),
        scratch_types=[
            pltpu.VMEM((per_w,), jnp.int32),
            pltpu.VMEM((ch, d), jnp.float32),
            pltpu.VMEM((ch, d), jnp.float32),
            pltpu.SemaphoreType.DMA,
            pltpu.SemaphoreType.DMA,
            pltpu.SemaphoreType.DMA,
            pltpu.SemaphoreType.DMA,
        ],
    )
    def gather_kernel(table_hbm, idx_hbm, out_hbm, idx_v, buf0, buf1,
                      gs0, gs1, ws0, ws1):
        wid = lax.axis_index("s") * n_cores + lax.axis_index("c")
        base = wid * per_w
        pltpu.sync_copy(idx_hbm.at[pl.ds(base, per_w)], idx_v)
        bufs = (buf0, buf1)
        gsems = (gs0, gs1)
        wsems = (ws0, ws1)
        gathers = [None] * n_ch
        for c in range(min(2, n_ch)):
            gathers[c] = pltpu.async_copy(
                table_hbm.at[idx_v.at[pl.ds(c * ch, ch)]], bufs[c % 2],
                gsems[c % 2])
        writes = []
        for c in range(n_ch):
            gathers[c].wait()
            if len(writes) >= 2:
                writes[c - 2].wait()
            writes.append(pltpu.async_copy(
                bufs[c % 2], out_hbm.at[pl.ds(base + c * ch, ch)],
                wsems[c % 2]))
            if c + 2 < n_ch:
                writes[c].wait()
                gathers[c + 2] = pltpu.async_copy(
                    table_hbm.at[idx_v.at[pl.ds((c + 2) * ch, ch)]],
                    bufs[c % 2], gsems[c % 2])
        for w in writes[-2:]:
            w.wait()

    return gather_kernel(table, idx)


def _tc_body(tt_ref, at_ref, g_ref, pos_ref, te_ref, ae_ref, ga_ref, be_ref,
             o_ref):
    x = g_ref[...] + pos_ref[...][None]
    tt = tt_ref[:, 0, :][..., None]
    at = at_ref[:, 0, :][..., None]
    x = x + jnp.where(tt == 1, te_ref[1], te_ref[0])
    acc = jnp.zeros_like(x)
    for k in range(5):
        acc = acc + jnp.where(at == k, ae_ref[k], 0.0)
    x = x + acc
    mu = jnp.mean(x, axis=-1, keepdims=True)
    xc = x - mu
    var = jnp.mean(xc * xc, axis=-1, keepdims=True)
    o_ref[...] = xc * lax.rsqrt(var + _EPS) * ga_ref[0] + be_ref[0]


def _tc_body_aliased(prev_ref, *refs):
    del prev_ref
    _tc_body(*refs)


def _tc_finish_chunk(c, prev_out, gathered_c, pos_emb, tt_c, at_c, type_emb,
                     ans_emb, gamma2, beta2, b, s, d, sc):
    nsteps = sc // _T
    base = c * nsteps
    in_specs = [
        pl.BlockSpec((b, 1, _T), lambda j: (0, 0, j)),
        pl.BlockSpec((b, 1, _T), lambda j: (0, 0, j)),
        pl.BlockSpec((b, _T, d), lambda j: (0, j, 0)),
        pl.BlockSpec((_T, d), lambda j, _b=base: (_b + j, 0)),
        pl.BlockSpec(type_emb.shape, lambda j: (0, 0)),
        pl.BlockSpec(ans_emb.shape, lambda j: (0, 0)),
        pl.BlockSpec((1, d), lambda j: (0, 0)),
        pl.BlockSpec((1, d), lambda j: (0, 0)),
    ]
    out_spec = pl.BlockSpec((b, _T, d), lambda j, _b=base: (0, _b + j, 0))
    args = (tt_c, at_c, gathered_c.reshape(b, sc, d), pos_emb, type_emb,
            ans_emb, gamma2, beta2)
    if prev_out is None:
        return pl.pallas_call(
            _tc_body,
            grid=(nsteps,),
            in_specs=in_specs,
            out_specs=out_spec,
            out_shape=jax.ShapeDtypeStruct((b, s, d), jnp.float32),
        )(*args)
    return pl.pallas_call(
        _tc_body_aliased,
        grid=(nsteps,),
        in_specs=[pl.BlockSpec(memory_space=pl.ANY)] + in_specs,
        out_specs=out_spec,
        out_shape=jax.ShapeDtypeStruct((b, s, d), jnp.float32),
        input_output_aliases={0: 0},
    )(prev_out, *args)


def kernel(input_ids, token_type_ids, ans_type_ids, word_emb, pos_emb, type_emb,
           ans_emb, gamma, beta):
    b, s = input_ids.shape
    d = word_emb.shape[1]
    sc = s // _NCHUNK
    ids_c = input_ids.reshape(b, _NCHUNK, sc).transpose(1, 0, 2)
    tt_c = token_type_ids.reshape(b, _NCHUNK, 1, sc).transpose(1, 0, 2, 3)
    at_c = ans_type_ids.reshape(b, _NCHUNK, 1, sc).transpose(1, 0, 2, 3)
    gamma2 = gamma.reshape(1, d)
    beta2 = beta.reshape(1, d)
    gathered = [
        _sc_gather(word_emb, ids_c[c].reshape(b * sc).astype(jnp.int32))
        for c in range(_NCHUNK)
    ]
    out = None
    for c in range(_NCHUNK):
        out = _tc_finish_chunk(c, out, gathered[c], pos_emb,
                               tt_c[c].astype(jnp.int32),
                               at_c[c].astype(jnp.int32),
                               type_emb, ans_emb, gamma2, beta2, b, s, d, sc)
    return out
